# baseline (speedup 1.0000x reference)
import functools

import jax
import jax.numpy as jnp
from jax import lax
from jax.experimental import pallas as pl
from jax.experimental.pallas import tpu as pltpu
from jax.experimental.pallas import tpu_sc as plsc

NC = 2
NS = 16
L = 16


def _ceil_to(x, m):
    return (x + m - 1) // m * m


def kernel(mem, idx, val):
    M, D = mem.shape
    B = idx.shape[0]
    K = 128
    CB = B // NS
    NQ = CB // K
    NP = 2
    R = _ceil_to(-(-M // (NC * NP)), K)

    mesh = plsc.VectorSubcoreMesh(core_axis_name="c", subcore_axis_name="s")

    @functools.partial(
        pl.kernel,
        mesh=mesh,
        compiler_params=pltpu.CompilerParams(use_tc_tiling_on_sc=False),
        out_type=jax.ShapeDtypeStruct((B + 8, D), jnp.float32),
        scratch_types=[
            pltpu.VMEM_SHARED((R + 8, D), jnp.float32),
            pltpu.VMEM((NQ, K), jnp.int32),
        ]
        + [pltpu.VMEM((K,), jnp.int32) for _ in range(NQ)]
        + [pltpu.VMEM((K,), jnp.int32) for _ in range(NQ)]
        + [
            pltpu.VMEM((K, D), jnp.float32),
            pltpu.VMEM((K, D), jnp.float32),
            pltpu.SemaphoreType.DMA,
        ],
    )
    def _k(mem_h, idx_h, val_h, out_h, acc, idxv, *rest):
        tgtv = rest[:NQ]
        jv = rest[NQ:2 * NQ]
        bufs = rest[2 * NQ:2 * NQ + 2]
        sem = rest[2 * NQ + 2]
        c = lax.axis_index("c")
        s = lax.axis_index("s")
        tb = s * CB

        pltpu.sync_copy(idx_h.at[pl.ds(s * NQ, NQ)], idxv)

        for p in range(NP):
            base = (p * NC + c) * R
            for q in range(NQ):
                for l in range(K // L):
                    v = idxv[q, pl.ds(l * L, L)]
                    m = (v >= base) & (v < base + R)
                    tgtv[q][pl.ds(l * L, L)] = jnp.where(m, v - base, R)
                    pos = tb + (q * K + l * L) + lax.iota(jnp.int32, L)
                    jv[q][pl.ds(l * L, L)] = jnp.where(m, pos, B)

            for q in range(NQ):
                b = bufs[q % 2]
                pltpu.async_copy(mem_h.at[idxv.at[q]], b, sem).wait()
                pltpu.async_copy(b, acc.at[tgtv[q]], sem).wait()
            plsc.subcore_barrier()
            for q in range(NQ):
                b = bufs[q % 2]
                pltpu.sync_copy(val_h.at[pl.ds(s * CB + q * K, K)], b)
                pltpu.sync_copy(b, acc.at[tgtv[q]], add=True)
            plsc.subcore_barrier()
            for q in range(NQ):
                b = bufs[q % 2]
                pltpu.async_copy(acc.at[tgtv[q]], b, sem).wait()
                pltpu.async_copy(b, out_h.at[jv[q]], sem).wait()
            plsc.subcore_barrier()

    idx2d = idx.astype(jnp.int32).reshape(NS * (B // NS // 128), 128)
    out_full = _k(mem, idx2d, val)
    return out_full[:B]


# speedup vs baseline: 1.0034x; 1.0034x over previous
import functools

import jax
import jax.numpy as jnp
from jax import lax
from jax.experimental import pallas as pl
from jax.experimental.pallas import tpu as pltpu
from jax.experimental.pallas import tpu_sc as plsc

NC = 2
NS = 16
L = 16


def _ceil_to(x, m):
    return (x + m - 1) // m * m


def kernel(mem, idx, val):
    M, D = mem.shape
    B = idx.shape[0]
    K = 128
    CB = B // NS
    NQ = CB // K
    NP = 2
    R = _ceil_to(-(-M // (NC * NP)), K)

    mesh = plsc.VectorSubcoreMesh(core_axis_name="c", subcore_axis_name="s")

    @functools.partial(
        pl.kernel,
        mesh=mesh,
        compiler_params=pltpu.CompilerParams(use_tc_tiling_on_sc=False),
        out_type=jax.ShapeDtypeStruct((B + 8, D), jnp.float32),
        scratch_types=[
            pltpu.VMEM_SHARED((R + 8, D), jnp.float32),
            pltpu.VMEM((NQ, K), jnp.int32),
        ]
        + [pltpu.VMEM((K,), jnp.int32) for _ in range(NQ)]
        + [pltpu.VMEM((K,), jnp.int32) for _ in range(NQ)]
        + [pltpu.VMEM((K, D), jnp.float32) for _ in range(3)]
        + [pltpu.SemaphoreType.DMA for _ in range(6)],
    )
    def _k(mem_h, idx_h, val_h, out_h, acc, idxv, *rest):
        NB = 3
        tgtv = rest[:NQ]
        jv = rest[NQ:2 * NQ]
        bufs = rest[2 * NQ:2 * NQ + NB]
        gsem = rest[2 * NQ + NB:2 * NQ + 2 * NB]
        ssem = rest[2 * NQ + 2 * NB:2 * NQ + 3 * NB]

        def stage(fire_gather, fire_scatter):
            gh = {}
            sh = {}
            for q in range(min(NB, NQ)):
                gh[q] = fire_gather(q, bufs[q % NB], gsem[q % NB])
            for q in range(NQ):
                sl = q % NB
                gh[q].wait()
                sh[q] = fire_scatter(q, bufs[sl], ssem[sl])
                if q + NB < NQ:
                    sh[q].wait()
                    gh[q + NB] = fire_gather(q + NB, bufs[sl], gsem[sl])
            for q in range(max(NQ - NB, 0), NQ):
                sh[q].wait()
        c = lax.axis_index("c")
        s = lax.axis_index("s")
        tb = s * CB

        pltpu.sync_copy(idx_h.at[pl.ds(s * NQ, NQ)], idxv)

        for p in range(NP):
            base = (p * NC + c) * R
            for q in range(NQ):
                for l in range(K // L):
                    v = idxv[q, pl.ds(l * L, L)]
                    m = (v >= base) & (v < base + R)
                    tgtv[q][pl.ds(l * L, L)] = jnp.where(m, v - base, R)
                    pos = tb + (q * K + l * L) + lax.iota(jnp.int32, L)
                    jv[q][pl.ds(l * L, L)] = jnp.where(m, pos, B)

            stage(lambda q, b, sm: pltpu.async_copy(mem_h.at[idxv.at[q]], b, sm),
                  lambda q, b, sm: pltpu.async_copy(b, acc.at[tgtv[q]], sm))
            plsc.subcore_barrier()
            stage(lambda q, b, sm: pltpu.async_copy(
                      val_h.at[pl.ds(s * CB + q * K, K)], b, sm),
                  lambda q, b, sm: pltpu.async_copy(
                      b, acc.at[tgtv[q]], sm, add=True))
            plsc.subcore_barrier()
            stage(lambda q, b, sm: pltpu.async_copy(acc.at[tgtv[q]], b, sm),
                  lambda q, b, sm: pltpu.async_copy(b, out_h.at[jv[q]], sm))
            plsc.subcore_barrier()

    idx2d = idx.astype(jnp.int32).reshape(NS * (B // NS // 128), 128)
    out_full = _k(mem, idx2d, val)
    return out_full[:B]
